# Initial kernel scaffold; baseline (speedup 1.0000x reference)
import jax
import jax.numpy as jnp
from jax.experimental import pallas as pl


def kernel(emb0, emb1, emb2, emb3, lap0_idx, lap0_val, lap1_idx, lap1_val, lap2_idx, lap2_val, lap3_idx, lap3_val, bnd1_idx, bnd1_val, bnd2_idx, bnd2_val, bnd3_idx, bnd3_val, W1s, W1d, W1u, W2s, W2d, W2u, W3s, W3d, W3u, lin1_W, lin1_b, rel_W, rel_b, rel_embed, order, idx, rel):
    raise NotImplementedError("write your pallas kernel here")


# speedup vs baseline: 2.8819x; 2.8819x over previous
import functools

import jax
import jax.numpy as jnp
from jax.experimental import pallas as pl

_N = 10000

_BLOCKS = (
    (0, 0, ('lap', 0), 0),
    (1, 0, ('lap', 1), 1),
    (2, 0, ('lap', 2), 2),
    (3, 0, ('lap', 3), 3),
    (1, 1, ('bndT', 1), 0),
    (2, 1, ('bndT', 2), 1),
    (3, 1, ('bndT', 3), 2),
    (0, 2, ('bnd', 1), 1),
    (1, 2, ('bnd', 2), 2),
    (2, 2, ('bnd', 3), 3),
)


def _mm_tanh_body(a_ref, w_ref, o_ref):
    o_ref[:] = jnp.tanh(
        jax.lax.dot_general(a_ref[:], w_ref[:], (((1,), (0,)), ((), ())),
                            preferred_element_type=jnp.float32))


def _mm_tanh(a, w, bm=400):
    m, k = a.shape
    n = w.shape[1]
    return pl.pallas_call(
        _mm_tanh_body,
        grid=(m // bm,),
        in_specs=[
            pl.BlockSpec((bm, k), lambda i: (i, 0)),
            pl.BlockSpec((k, n), lambda i: (0, 0)),
        ],
        out_specs=pl.BlockSpec((bm, n), lambda i: (i, 0)),
        out_shape=jax.ShapeDtypeStruct((m, n), jnp.float32),
    )(a, w)


def _spmm(dst, src, val, x, n_out):
    return jnp.zeros((n_out, x.shape[1]), x.dtype).at[dst].add(
        val[:, None] * x[src])


def _layer(embs, graphs, ws, wd, wu):
    w_in = embs.shape[-1]
    zs, zd, zu = [], [], []
    zero = jnp.zeros((_N, w_in), jnp.float32)
    per_order = {i: [zero, zero, zero] for i in range(4)}
    for (oi, slot, gkey, so) in _BLOCKS:
        dst, src, val = graphs[gkey]
        per_order[oi][slot] = _spmm(dst, src, val, embs[so], _N)
    a = jnp.stack([jnp.concatenate(per_order[i], axis=-1) for i in range(4)])
    wcat = jnp.concatenate([ws, wd, wu], axis=0)
    out = _mm_tanh(a.reshape(4 * _N, 3 * w_in), wcat)
    return out.reshape(4, _N, -1)


def kernel(emb0, emb1, emb2, emb3, lap0_idx, lap0_val, lap1_idx, lap1_val,
           lap2_idx, lap2_val, lap3_idx, lap3_val, bnd1_idx, bnd1_val,
           bnd2_idx, bnd2_val, bnd3_idx, bnd3_val, W1s, W1d, W1u, W2s, W2d,
           W2u, W3s, W3d, W3u, lin1_W, lin1_b, rel_W, rel_b, rel_embed,
           order, idx, rel):
    graphs = {}
    for i, (gi, gv) in enumerate([(lap0_idx, lap0_val), (lap1_idx, lap1_val),
                                  (lap2_idx, lap2_val), (lap3_idx, lap3_val)]):
        graphs[('lap', i)] = (gi[0], gi[1], gv)
    for i, (gi, gv) in enumerate([(bnd1_idx, bnd1_val), (bnd2_idx, bnd2_val),
                                  (bnd3_idx, bnd3_val)], start=1):
        graphs[('bnd', i)] = (gi[0], gi[1], gv)
        graphs[('bndT', i)] = (gi[1], gi[0], gv)

    e0 = jnp.stack([emb0, emb1, emb2, emb3])
    e1 = _layer(e0, graphs, W1s, W1d, W1u)
    e2 = _layer(e1, graphs, W2s, W2d, W2u)

    tvecs = []
    masks = []
    for (oi, slot, gkey, so) in _BLOCKS:
        dst, src, val = graphs[gkey]
        wv = jnp.zeros((_N,), jnp.float32).at[src].add(
            jnp.where(dst == idx, val, 0.0))
        tvecs.append(wv @ e2[so])
        masks.append((order == oi).astype(jnp.float32))
    t = jnp.stack(tvecs)
    m = jnp.stack(masks)
    tm = t * m[:, None]
    xs = tm[0] + tm[1] + tm[2] + tm[3]
    xd = tm[4] + tm[5] + tm[6]
    xu = tm[7] + tm[8] + tm[9]
    h = xs @ W3s + xd @ W3d + xu @ W3u
    e3row = jnp.tanh(h)
    final = jnp.tanh(e3row @ lin1_W + lin1_b)
    s0 = final @ rel_W[:final.shape[0], 0]
    scores = s0 + rel_embed @ rel_W[final.shape[0]:, 0] + rel_b[0]
    nz = jnp.nonzero(rel, size=rel.shape[0])[0][:, None]
    return scores[nz]
